# Initial kernel scaffold; baseline (speedup 1.0000x reference)
import jax
import jax.numpy as jnp
from jax.experimental import pallas as pl


def kernel(x, atomic_numbers, count_sketches):
    raise NotImplementedError("write your pallas kernel here")


# speedup vs baseline: 13.9392x; 13.9392x over previous
import functools

import jax
import jax.numpy as jnp
from jax.experimental import pallas as pl
from jax.experimental.pallas import tpu as pltpu

_DEGREE = 2
_NRF = 512
_IN_DIM = 128
_NUM_SPECIES = 8
_LENGTH_SCALE = 1.0
_TILE = 2000


def _dot(a, b):
    return jax.lax.dot_general(
        a, b, (((1,), (0,)), ((), ())),
        precision=jax.lax.Precision.HIGHEST,
        preferred_element_type=jnp.float32,
    )


def _dot_contract00(a, b):
    return jax.lax.dot_general(
        a, b, (((0,), (0,)), ((), ())),
        precision=jax.lax.Precision.HIGHEST,
        preferred_element_type=jnp.float32,
    )


def _sketch_kernel(x_ref, an_ref, cs_ref, ecos_ref, esin_ref, out_ref,
                   b0re, b0im, b1re, b1im, accre, accim, counts):
    i = pl.program_id(0)
    nsteps = pl.num_programs(0)

    @pl.when(i == 0)
    def _init():
        b0re[...] = _dot_contract00(cs_ref[0], ecos_ref[...])
        b0im[...] = -_dot_contract00(cs_ref[0], esin_ref[...])
        b1re[...] = _dot_contract00(cs_ref[1], ecos_ref[...])
        b1im[...] = -_dot_contract00(cs_ref[1], esin_ref[...])
        accre[...] = jnp.zeros_like(accre)
        accim[...] = jnp.zeros_like(accim)
        counts[...] = jnp.zeros_like(counts)

    xt = x_ref[...] * (1.0 / _LENGTH_SCALE)
    f0re = _dot(xt, b0re[...])
    f0im = _dot(xt, b0im[...])
    f1re = _dot(xt, b1re[...])
    f1im = _dot(xt, b1im[...])

    pre = f0re * f1re - f0im * f1im
    pim = f0re * f1im + f0im * f1re

    an = an_ref[0]
    sp = jax.lax.broadcasted_iota(jnp.int32, (_NUM_SPECIES, an.shape[1]), 0)
    onehot = (sp == an).astype(jnp.float32)
    accre[...] += _dot(onehot, pre)
    accim[...] += _dot(onehot, pim)
    counts[...] += jnp.sum(onehot, axis=1, keepdims=True)

    @pl.when(i == nsteps - 1)
    def _fini():
        z = _dot(accre[...], ecos_ref[...]) - _dot(accim[...], esin_ref[...])
        denom = (counts[...][:, :1] + 1.0) * (_NUM_SPECIES * _NRF)
        out_ref[...] = z / denom


@functools.partial(jax.jit, static_argnames=())
def _run(x, an3, count_sketches, ecos, esin):
    n = x.shape[0]
    nsteps = n // _TILE
    grid = (nsteps,)
    out = pl.pallas_call(
        _sketch_kernel,
        grid=grid,
        in_specs=[
            pl.BlockSpec((_TILE, _IN_DIM), lambda i: (i, 0)),
            pl.BlockSpec((1, 1, _TILE), lambda i: (i, 0, 0)),
            pl.BlockSpec((_DEGREE, _NRF, _IN_DIM), lambda i: (0, 0, 0)),
            pl.BlockSpec((_NRF, _NRF), lambda i: (0, 0)),
            pl.BlockSpec((_NRF, _NRF), lambda i: (0, 0)),
        ],
        out_specs=pl.BlockSpec((_NUM_SPECIES, _NRF), lambda i: (0, 0)),
        out_shape=jax.ShapeDtypeStruct((_NUM_SPECIES, _NRF), jnp.float32),
        scratch_shapes=[
            pltpu.VMEM((_IN_DIM, _NRF), jnp.float32),
            pltpu.VMEM((_IN_DIM, _NRF), jnp.float32),
            pltpu.VMEM((_IN_DIM, _NRF), jnp.float32),
            pltpu.VMEM((_IN_DIM, _NRF), jnp.float32),
            pltpu.VMEM((_NUM_SPECIES, _NRF), jnp.float32),
            pltpu.VMEM((_NUM_SPECIES, _NRF), jnp.float32),
            pltpu.VMEM((_NUM_SPECIES, 128), jnp.float32),
        ],
    )(x, an3, count_sketches, ecos, esin)
    return out.reshape(-1)


def kernel(x, atomic_numbers, count_sketches):
    n = x.shape[0]
    r = jnp.arange(_NRF, dtype=jnp.int32)
    m = (r[:, None] * r[None, :]) % _NRF
    ang = m.astype(jnp.float32) * (2.0 * jnp.pi / _NRF)
    ecos = jnp.cos(ang)
    esin = jnp.sin(ang)
    an3 = atomic_numbers.astype(jnp.int32).reshape(n // _TILE, 1, _TILE)
    return _run(x, an3, count_sketches, ecos, esin)


# speedup vs baseline: 66.0001x; 4.7349x over previous
import functools

import jax
import jax.numpy as jnp
from jax.experimental import pallas as pl
from jax.experimental.pallas import tpu as pltpu

_DEGREE = 2
_NRF = 512
_IN_DIM = 128
_NUM_SPECIES = 8
_LENGTH_SCALE = 1.0
_TILE = 2000


def _dot(a, b):
    return jax.lax.dot_general(
        a, b, (((1,), (0,)), ((), ())),
        precision=jax.lax.Precision.DEFAULT,
        preferred_element_type=jnp.float32,
    )


def _dot_contract00(a, b):
    return jax.lax.dot_general(
        a, b, (((0,), (0,)), ((), ())),
        precision=jax.lax.Precision.HIGHEST,
        preferred_element_type=jnp.float32,
    )


def _sketch_kernel(x_ref, an_ref, cs_ref, ecos_ref, esin_ref, out_ref,
                   b0re, b0im, b1re, b1im, accre, accim, counts):
    i = pl.program_id(0)
    nsteps = pl.num_programs(0)

    @pl.when(i == 0)
    def _init():
        b0re[...] = _dot_contract00(cs_ref[0], ecos_ref[...])
        b0im[...] = -_dot_contract00(cs_ref[0], esin_ref[...])
        b1re[...] = _dot_contract00(cs_ref[1], ecos_ref[...])
        b1im[...] = -_dot_contract00(cs_ref[1], esin_ref[...])
        accre[...] = jnp.zeros_like(accre)
        accim[...] = jnp.zeros_like(accim)
        counts[...] = jnp.zeros_like(counts)

    xt = x_ref[...] * (1.0 / _LENGTH_SCALE)
    f0re = _dot(xt, b0re[...])
    f0im = _dot(xt, b0im[...])
    f1re = _dot(xt, b1re[...])
    f1im = _dot(xt, b1im[...])

    pre = f0re * f1re - f0im * f1im
    pim = f0re * f1im + f0im * f1re

    an = an_ref[0]
    sp = jax.lax.broadcasted_iota(jnp.int32, (_NUM_SPECIES, an.shape[1]), 0)
    onehot = (sp == an).astype(jnp.float32)
    accre[...] += _dot(onehot, pre)
    accim[...] += _dot(onehot, pim)
    counts[...] += jnp.sum(onehot, axis=1, keepdims=True)

    @pl.when(i == nsteps - 1)
    def _fini():
        z = _dot(accre[...], ecos_ref[...]) - _dot(accim[...], esin_ref[...])
        denom = (counts[...][:, :1] + 1.0) * (_NUM_SPECIES * _NRF)
        out_ref[...] = z / denom


@functools.partial(jax.jit, static_argnames=())
def _run(x, an3, count_sketches, ecos, esin):
    n = x.shape[0]
    nsteps = n // _TILE
    grid = (nsteps,)
    out = pl.pallas_call(
        _sketch_kernel,
        grid=grid,
        in_specs=[
            pl.BlockSpec((_TILE, _IN_DIM), lambda i: (i, 0)),
            pl.BlockSpec((1, 1, _TILE), lambda i: (i, 0, 0)),
            pl.BlockSpec((_DEGREE, _NRF, _IN_DIM), lambda i: (0, 0, 0)),
            pl.BlockSpec((_NRF, _NRF), lambda i: (0, 0)),
            pl.BlockSpec((_NRF, _NRF), lambda i: (0, 0)),
        ],
        out_specs=pl.BlockSpec((_NUM_SPECIES, _NRF), lambda i: (0, 0)),
        out_shape=jax.ShapeDtypeStruct((_NUM_SPECIES, _NRF), jnp.float32),
        scratch_shapes=[
            pltpu.VMEM((_IN_DIM, _NRF), jnp.float32),
            pltpu.VMEM((_IN_DIM, _NRF), jnp.float32),
            pltpu.VMEM((_IN_DIM, _NRF), jnp.float32),
            pltpu.VMEM((_IN_DIM, _NRF), jnp.float32),
            pltpu.VMEM((_NUM_SPECIES, _NRF), jnp.float32),
            pltpu.VMEM((_NUM_SPECIES, _NRF), jnp.float32),
            pltpu.VMEM((_NUM_SPECIES, 128), jnp.float32),
        ],
    )(x, an3, count_sketches, ecos, esin)
    return out.reshape(-1)


def kernel(x, atomic_numbers, count_sketches):
    n = x.shape[0]
    r = jnp.arange(_NRF, dtype=jnp.int32)
    m = (r[:, None] * r[None, :]) % _NRF
    ang = m.astype(jnp.float32) * (2.0 * jnp.pi / _NRF)
    ecos = jnp.cos(ang)
    esin = jnp.sin(ang)
    an3 = atomic_numbers.astype(jnp.int32).reshape(n // _TILE, 1, _TILE)
    return _run(x, an3, count_sketches, ecos, esin)


# speedup vs baseline: 99.2309x; 1.5035x over previous
import functools

import jax
import jax.numpy as jnp
from jax.experimental import pallas as pl
from jax.experimental.pallas import tpu as pltpu

_DEGREE = 2
_NRF = 512
_KH = _NRF // 2
_IN_DIM = 128
_NUM_SPECIES = 8
_LENGTH_SCALE = 1.0
_TILE = 2000


def _dot(a, b, precision=jax.lax.Precision.DEFAULT):
    return jax.lax.dot_general(
        a, b, (((1,), (0,)), ((), ())),
        precision=precision,
        preferred_element_type=jnp.float32,
    )


def _dot_c00(a, b):
    return jax.lax.dot_general(
        a, b, (((0,), (0,)), ((), ())),
        precision=jax.lax.Precision.HIGHEST,
        preferred_element_type=jnp.float32,
    )


def _sketch_kernel(x_ref, an_ref, cs_ref, ecb_ref, esb_ref, ecf_ref, esf_ref,
                   out_ref,
                   b0re, b0im, b1re, b1im, bn, acc, nyq, counts):
    i = pl.program_id(0)
    nsteps = pl.num_programs(0)

    @pl.when(i == 0)
    def _init():
        b0re[...] = _dot_c00(cs_ref[0], ecb_ref[...])
        b0im[...] = -_dot_c00(cs_ref[0], esb_ref[...])
        b1re[...] = _dot_c00(cs_ref[1], ecb_ref[...])
        b1im[...] = -_dot_c00(cs_ref[1], esb_ref[...])
        r = jax.lax.broadcasted_iota(jnp.int32, (_NRF, 1), 0)
        alt = jnp.where(r % 2 == 0, 1.0, -1.0).astype(jnp.float32)
        bn[...] = jnp.concatenate(
            [jnp.sum(cs_ref[0] * alt, axis=0, keepdims=True),
             jnp.sum(cs_ref[1] * alt, axis=0, keepdims=True)], axis=0)
        acc[...] = jnp.zeros_like(acc)
        nyq[...] = jnp.zeros_like(nyq)
        counts[...] = jnp.zeros_like(counts)

    xt = x_ref[...] * (1.0 / _LENGTH_SCALE)
    f0re = _dot(xt, b0re[...])
    f0im = _dot(xt, b0im[...])
    f1re = _dot(xt, b1re[...])
    f1im = _dot(xt, b1im[...])

    pre = f0re * f1re - f0im * f1im
    pim = f0re * f1im + f0im * f1re
    pp = jnp.concatenate([pre, pim], axis=1)

    fn0 = jnp.sum(xt * bn[0:1, :], axis=1, keepdims=True)
    fn1 = jnp.sum(xt * bn[1:2, :], axis=1, keepdims=True)
    pn = fn0 * fn1

    an = an_ref[0]
    sp = jax.lax.broadcasted_iota(jnp.int32, (_NUM_SPECIES, an.shape[1]), 0)
    onehot = (sp == an).astype(jnp.float32)
    acc[...] += _dot(onehot, pp)
    nyq[...] += _dot(onehot, pn)
    counts[...] += jnp.sum(onehot, axis=1, keepdims=True)

    @pl.when(i == nsteps - 1)
    def _fini():
        k = jax.lax.broadcasted_iota(jnp.int32, (_NUM_SPECIES, _KH), 1)
        w = jnp.where(k == 0, 1.0, 2.0).astype(jnp.float32)
        are = acc[:, :_KH] * w
        aim = acc[:, _KH:] * w
        t = jax.lax.broadcasted_iota(jnp.int32, (1, _NRF), 1)
        alt_t = jnp.where(t % 2 == 0, 1.0, -1.0).astype(jnp.float32)
        z = (_dot(are, ecf_ref[...], jax.lax.Precision.HIGHEST)
             - _dot(aim, esf_ref[...], jax.lax.Precision.HIGHEST)
             + nyq[:, 0:1] * alt_t)
        denom = (counts[:, 0:1] + 1.0) * float(_NUM_SPECIES * _NRF)
        out_ref[...] = z / denom


@jax.jit
def _run(x, an3, count_sketches, ecb, esb, ecf, esf):
    n = x.shape[0]
    nsteps = n // _TILE
    out = pl.pallas_call(
        _sketch_kernel,
        grid=(nsteps,),
        in_specs=[
            pl.BlockSpec((_TILE, _IN_DIM), lambda i: (i, 0)),
            pl.BlockSpec((1, 1, _TILE), lambda i: (i, 0, 0)),
            pl.BlockSpec((_DEGREE, _NRF, _IN_DIM), lambda i: (0, 0, 0)),
            pl.BlockSpec((_NRF, _KH), lambda i: (0, 0)),
            pl.BlockSpec((_NRF, _KH), lambda i: (0, 0)),
            pl.BlockSpec((_KH, _NRF), lambda i: (0, 0)),
            pl.BlockSpec((_KH, _NRF), lambda i: (0, 0)),
        ],
        out_specs=pl.BlockSpec((_NUM_SPECIES, _NRF), lambda i: (0, 0)),
        out_shape=jax.ShapeDtypeStruct((_NUM_SPECIES, _NRF), jnp.float32),
        scratch_shapes=[
            pltpu.VMEM((_IN_DIM, _KH), jnp.float32),
            pltpu.VMEM((_IN_DIM, _KH), jnp.float32),
            pltpu.VMEM((_IN_DIM, _KH), jnp.float32),
            pltpu.VMEM((_IN_DIM, _KH), jnp.float32),
            pltpu.VMEM((_DEGREE, _IN_DIM), jnp.float32),
            pltpu.VMEM((_NUM_SPECIES, _NRF), jnp.float32),
            pltpu.VMEM((_NUM_SPECIES, 128), jnp.float32),
            pltpu.VMEM((_NUM_SPECIES, 128), jnp.float32),
        ],
    )(x, an3, count_sketches, ecb, esb, ecf, esf)
    return out.reshape(-1)


def kernel(x, atomic_numbers, count_sketches):
    n = x.shape[0]
    r = jnp.arange(_NRF, dtype=jnp.int32)
    m = (r[:, None] * r[None, :]) % _NRF
    ang = m.astype(jnp.float32) * (2.0 * jnp.pi / _NRF)
    ecos = jnp.cos(ang)
    esin = jnp.sin(ang)
    ecb, esb = ecos[:, :_KH], esin[:, :_KH]
    ecf, esf = ecos[:_KH, :], esin[:_KH, :]
    an3 = atomic_numbers.astype(jnp.int32).reshape(n // _TILE, 1, _TILE)
    return _run(x, an3, count_sketches, ecb, esb, ecf, esf)


# speedup vs baseline: 100.8016x; 1.0158x over previous
import functools

import jax
import jax.numpy as jnp
from jax.experimental import pallas as pl
from jax.experimental.pallas import tpu as pltpu

_DEGREE = 2
_NRF = 512
_KH = _NRF // 2
_IN_DIM = 128
_NUM_SPECIES = 8
_LENGTH_SCALE = 1.0
_TILE = 5000


def _dot(a, b, precision=jax.lax.Precision.DEFAULT):
    return jax.lax.dot_general(
        a, b, (((1,), (0,)), ((), ())),
        precision=precision,
        preferred_element_type=jnp.float32,
    )


def _dot_c00(a, b):
    return jax.lax.dot_general(
        a, b, (((0,), (0,)), ((), ())),
        precision=jax.lax.Precision.HIGHEST,
        preferred_element_type=jnp.float32,
    )


def _sketch_kernel(x_ref, an_ref, cs_ref, ecb_ref, esb_ref, ecf_ref, esf_ref,
                   out_ref,
                   b0re, b0im, b1re, b1im, bn, acc, nyq, counts):
    i = pl.program_id(0)
    nsteps = pl.num_programs(0)

    @pl.when(i == 0)
    def _init():
        b0re[...] = _dot_c00(cs_ref[0], ecb_ref[...])
        b0im[...] = -_dot_c00(cs_ref[0], esb_ref[...])
        b1re[...] = _dot_c00(cs_ref[1], ecb_ref[...])
        b1im[...] = -_dot_c00(cs_ref[1], esb_ref[...])
        r = jax.lax.broadcasted_iota(jnp.int32, (_NRF, 1), 0)
        alt = jnp.where(r % 2 == 0, 1.0, -1.0).astype(jnp.float32)
        bn[...] = jnp.concatenate(
            [jnp.sum(cs_ref[0] * alt, axis=0, keepdims=True),
             jnp.sum(cs_ref[1] * alt, axis=0, keepdims=True)], axis=0)
        acc[...] = jnp.zeros_like(acc)
        nyq[...] = jnp.zeros_like(nyq)
        counts[...] = jnp.zeros_like(counts)

    xt = x_ref[...] * (1.0 / _LENGTH_SCALE)
    f0re = _dot(xt, b0re[...])
    f0im = _dot(xt, b0im[...])
    f1re = _dot(xt, b1re[...])
    f1im = _dot(xt, b1im[...])

    pre = f0re * f1re - f0im * f1im
    pim = f0re * f1im + f0im * f1re

    fn0 = jnp.sum(xt * bn[0:1, :], axis=1, keepdims=True)
    fn1 = jnp.sum(xt * bn[1:2, :], axis=1, keepdims=True)
    pn = fn0 * fn1

    an = an_ref[0]
    sp = jax.lax.broadcasted_iota(jnp.int32, (_NUM_SPECIES, an.shape[1]), 0)
    onehot = (sp == an).astype(jnp.float32)
    acc[:, :_KH] += _dot(onehot, pre)
    acc[:, _KH:] += _dot(onehot, pim)
    nyq[...] += _dot(onehot, pn)
    counts[...] += jnp.sum(onehot, axis=1, keepdims=True)

    @pl.when(i == nsteps - 1)
    def _fini():
        k = jax.lax.broadcasted_iota(jnp.int32, (_NUM_SPECIES, _KH), 1)
        w = jnp.where(k == 0, 1.0, 2.0).astype(jnp.float32)
        are = acc[:, :_KH] * w
        aim = acc[:, _KH:] * w
        t = jax.lax.broadcasted_iota(jnp.int32, (1, _NRF), 1)
        alt_t = jnp.where(t % 2 == 0, 1.0, -1.0).astype(jnp.float32)
        z = (_dot(are, ecf_ref[...], jax.lax.Precision.HIGHEST)
             - _dot(aim, esf_ref[...], jax.lax.Precision.HIGHEST)
             + nyq[:, 0:1] * alt_t)
        denom = (counts[:, 0:1] + 1.0) * float(_NUM_SPECIES * _NRF)
        out_ref[...] = z / denom


@jax.jit
def _run(x, an3, count_sketches, ecb, esb, ecf, esf):
    n = x.shape[0]
    nsteps = n // _TILE
    out = pl.pallas_call(
        _sketch_kernel,
        grid=(nsteps,),
        in_specs=[
            pl.BlockSpec((_TILE, _IN_DIM), lambda i: (i, 0)),
            pl.BlockSpec((1, 1, _TILE), lambda i: (i, 0, 0)),
            pl.BlockSpec((_DEGREE, _NRF, _IN_DIM), lambda i: (0, 0, 0)),
            pl.BlockSpec((_NRF, _KH), lambda i: (0, 0)),
            pl.BlockSpec((_NRF, _KH), lambda i: (0, 0)),
            pl.BlockSpec((_KH, _NRF), lambda i: (0, 0)),
            pl.BlockSpec((_KH, _NRF), lambda i: (0, 0)),
        ],
        out_specs=pl.BlockSpec((_NUM_SPECIES, _NRF), lambda i: (0, 0)),
        out_shape=jax.ShapeDtypeStruct((_NUM_SPECIES, _NRF), jnp.float32),
        scratch_shapes=[
            pltpu.VMEM((_IN_DIM, _KH), jnp.float32),
            pltpu.VMEM((_IN_DIM, _KH), jnp.float32),
            pltpu.VMEM((_IN_DIM, _KH), jnp.float32),
            pltpu.VMEM((_IN_DIM, _KH), jnp.float32),
            pltpu.VMEM((_DEGREE, _IN_DIM), jnp.float32),
            pltpu.VMEM((_NUM_SPECIES, _NRF), jnp.float32),
            pltpu.VMEM((_NUM_SPECIES, 128), jnp.float32),
            pltpu.VMEM((_NUM_SPECIES, 128), jnp.float32),
        ],
    )(x, an3, count_sketches, ecb, esb, ecf, esf)
    return out.reshape(-1)


def kernel(x, atomic_numbers, count_sketches):
    n = x.shape[0]
    r = jnp.arange(_NRF, dtype=jnp.int32)
    m = (r[:, None] * r[None, :]) % _NRF
    ang = m.astype(jnp.float32) * (2.0 * jnp.pi / _NRF)
    ecos = jnp.cos(ang)
    esin = jnp.sin(ang)
    ecb, esb = ecos[:, :_KH], esin[:, :_KH]
    ecf, esf = ecos[:_KH, :], esin[:_KH, :]
    an3 = atomic_numbers.astype(jnp.int32).reshape(n // _TILE, 1, _TILE)
    return _run(x, an3, count_sketches, ecb, esb, ecf, esf)


# speedup vs baseline: 101.4203x; 1.0061x over previous
import functools

import jax
import jax.numpy as jnp
from jax.experimental import pallas as pl
from jax.experimental.pallas import tpu as pltpu

_DEGREE = 2
_NRF = 512
_KH = _NRF // 2
_IN_DIM = 128
_NUM_SPECIES = 8
_LENGTH_SCALE = 1.0
_TILE = 5000


def _dot(a, b, precision=jax.lax.Precision.DEFAULT):
    return jax.lax.dot_general(
        a, b, (((1,), (0,)), ((), ())),
        precision=precision,
        preferred_element_type=jnp.float32,
    )


def _dot_c00(a, b):
    return jax.lax.dot_general(
        a, b, (((0,), (0,)), ((), ())),
        precision=jax.lax.Precision.HIGHEST,
        preferred_element_type=jnp.float32,
    )


def _sketch_kernel(x_ref, an_ref, cs_ref, ecb_ref, esb_ref, ecf_ref, esf_ref,
                   out_ref,
                   b0re, b0im, b1re, b1im, bn, acc, nyq, counts):
    i = pl.program_id(0)
    nsteps = pl.num_programs(0)

    @pl.when(i == 0)
    def _init():
        b0re[...] = _dot_c00(cs_ref[0], ecb_ref[...]).astype(jnp.bfloat16)
        b0im[...] = (-_dot_c00(cs_ref[0], esb_ref[...])).astype(jnp.bfloat16)
        b1re[...] = _dot_c00(cs_ref[1], ecb_ref[...]).astype(jnp.bfloat16)
        b1im[...] = (-_dot_c00(cs_ref[1], esb_ref[...])).astype(jnp.bfloat16)
        r = jax.lax.broadcasted_iota(jnp.int32, (_NRF, 1), 0)
        alt = jnp.where(r % 2 == 0, 1.0, -1.0).astype(jnp.float32)
        bn[...] = jnp.concatenate(
            [jnp.sum(cs_ref[0] * alt, axis=0, keepdims=True),
             jnp.sum(cs_ref[1] * alt, axis=0, keepdims=True)], axis=0)
        acc[...] = jnp.zeros_like(acc)
        nyq[...] = jnp.zeros_like(nyq)
        counts[...] = jnp.zeros_like(counts)

    xt = x_ref[...] * (1.0 / _LENGTH_SCALE)
    xb = xt.astype(jnp.bfloat16)
    f0re = _dot(xb, b0re[...])
    f0im = _dot(xb, b0im[...])
    f1re = _dot(xb, b1re[...])
    f1im = _dot(xb, b1im[...])

    pre = (f0re * f1re - f0im * f1im).astype(jnp.bfloat16)
    pim = (f0re * f1im + f0im * f1re).astype(jnp.bfloat16)

    fn0 = jnp.sum(xt * bn[0:1, :], axis=1, keepdims=True)
    fn1 = jnp.sum(xt * bn[1:2, :], axis=1, keepdims=True)
    pn = fn0 * fn1

    an = an_ref[0]
    sp = jax.lax.broadcasted_iota(jnp.int32, (_NUM_SPECIES, an.shape[1]), 0)
    onehot = (sp == an).astype(jnp.bfloat16)
    acc[:, :_KH] += _dot(onehot, pre)
    acc[:, _KH:] += _dot(onehot, pim)
    nyq[...] += _dot(onehot.astype(jnp.float32), pn)
    counts[...] += jnp.sum(onehot.astype(jnp.float32), axis=1, keepdims=True)

    @pl.when(i == nsteps - 1)
    def _fini():
        k = jax.lax.broadcasted_iota(jnp.int32, (_NUM_SPECIES, _KH), 1)
        w = jnp.where(k == 0, 1.0, 2.0).astype(jnp.float32)
        are = acc[:, :_KH] * w
        aim = acc[:, _KH:] * w
        t = jax.lax.broadcasted_iota(jnp.int32, (1, _NRF), 1)
        alt_t = jnp.where(t % 2 == 0, 1.0, -1.0).astype(jnp.float32)
        z = (_dot(are, ecf_ref[...], jax.lax.Precision.HIGHEST)
             - _dot(aim, esf_ref[...], jax.lax.Precision.HIGHEST)
             + nyq[:, 0:1] * alt_t)
        denom = (counts[:, 0:1] + 1.0) * float(_NUM_SPECIES * _NRF)
        out_ref[...] = z / denom


@jax.jit
def _run(x, an3, count_sketches, ecb, esb, ecf, esf):
    n = x.shape[0]
    nsteps = n // _TILE
    out = pl.pallas_call(
        _sketch_kernel,
        grid=(nsteps,),
        in_specs=[
            pl.BlockSpec((_TILE, _IN_DIM), lambda i: (i, 0)),
            pl.BlockSpec((1, 1, _TILE), lambda i: (i, 0, 0)),
            pl.BlockSpec((_DEGREE, _NRF, _IN_DIM), lambda i: (0, 0, 0)),
            pl.BlockSpec((_NRF, _KH), lambda i: (0, 0)),
            pl.BlockSpec((_NRF, _KH), lambda i: (0, 0)),
            pl.BlockSpec((_KH, _NRF), lambda i: (0, 0)),
            pl.BlockSpec((_KH, _NRF), lambda i: (0, 0)),
        ],
        out_specs=pl.BlockSpec((_NUM_SPECIES, _NRF), lambda i: (0, 0)),
        out_shape=jax.ShapeDtypeStruct((_NUM_SPECIES, _NRF), jnp.float32),
        scratch_shapes=[
            pltpu.VMEM((_IN_DIM, _KH), jnp.bfloat16),
            pltpu.VMEM((_IN_DIM, _KH), jnp.bfloat16),
            pltpu.VMEM((_IN_DIM, _KH), jnp.bfloat16),
            pltpu.VMEM((_IN_DIM, _KH), jnp.bfloat16),
            pltpu.VMEM((_DEGREE, _IN_DIM), jnp.float32),
            pltpu.VMEM((_NUM_SPECIES, _NRF), jnp.float32),
            pltpu.VMEM((_NUM_SPECIES, 128), jnp.float32),
            pltpu.VMEM((_NUM_SPECIES, 128), jnp.float32),
        ],
    )(x, an3, count_sketches, ecb, esb, ecf, esf)
    return out.reshape(-1)


def kernel(x, atomic_numbers, count_sketches):
    n = x.shape[0]
    r = jnp.arange(_NRF, dtype=jnp.int32)
    m = (r[:, None] * r[None, :]) % _NRF
    ang = m.astype(jnp.float32) * (2.0 * jnp.pi / _NRF)
    ecos = jnp.cos(ang)
    esin = jnp.sin(ang)
    ecb, esb = ecos[:, :_KH], esin[:, :_KH]
    ecf, esf = ecos[:_KH, :], esin[:_KH, :]
    an3 = atomic_numbers.astype(jnp.int32).reshape(n // _TILE, 1, _TILE)
    return _run(x, an3, count_sketches, ecb, esb, ecf, esf)


# speedup vs baseline: 101.5658x; 1.0014x over previous
import functools

import jax
import jax.numpy as jnp
from jax.experimental import pallas as pl
from jax.experimental.pallas import tpu as pltpu

_DEGREE = 2
_NRF = 512
_KH = _NRF // 2
_IN_DIM = 128
_NUM_SPECIES = 8
_LENGTH_SCALE = 1.0
_TILE = 5000


def _dot(a, b, precision=jax.lax.Precision.DEFAULT,
         out_dtype=jnp.float32):
    return jax.lax.dot_general(
        a, b, (((1,), (0,)), ((), ())),
        precision=precision,
        preferred_element_type=out_dtype,
    )


def _dot_c00(a, b):
    return jax.lax.dot_general(
        a, b, (((0,), (0,)), ((), ())),
        precision=jax.lax.Precision.HIGHEST,
        preferred_element_type=jnp.float32,
    )


def _sketch_kernel(x_ref, an_ref, cs_ref, ecb_ref, esb_ref, ecf_ref, esf_ref,
                   out_ref,
                   b0re, b0im, b1re, b1im, bn, acc, nyq, counts):
    i = pl.program_id(0)
    nsteps = pl.num_programs(0)

    @pl.when(i == 0)
    def _init():
        b0re[...] = _dot_c00(cs_ref[0], ecb_ref[...]).astype(jnp.bfloat16)
        b0im[...] = (-_dot_c00(cs_ref[0], esb_ref[...])).astype(jnp.bfloat16)
        b1re[...] = _dot_c00(cs_ref[1], ecb_ref[...]).astype(jnp.bfloat16)
        b1im[...] = (-_dot_c00(cs_ref[1], esb_ref[...])).astype(jnp.bfloat16)
        r = jax.lax.broadcasted_iota(jnp.int32, (_NRF, 1), 0)
        alt = jnp.where(r % 2 == 0, 1.0, -1.0).astype(jnp.float32)
        bn[...] = jnp.concatenate(
            [jnp.sum(cs_ref[0] * alt, axis=0, keepdims=True),
             jnp.sum(cs_ref[1] * alt, axis=0, keepdims=True)], axis=0)
        acc[...] = jnp.zeros_like(acc)
        nyq[...] = jnp.zeros_like(nyq)
        counts[...] = jnp.zeros_like(counts)

    xt = x_ref[...] * (1.0 / _LENGTH_SCALE)
    xb = xt.astype(jnp.bfloat16)
    f0re = _dot(xb, b0re[...])
    f0im = _dot(xb, b0im[...])
    f1re = _dot(xb, b1re[...])
    f1im = _dot(xb, b1im[...])

    pre = (f0re * f1re - f0im * f1im).astype(jnp.bfloat16)
    pim = (f0re * f1im + f0im * f1re).astype(jnp.bfloat16)

    fn0 = jnp.sum(xt * bn[0:1, :], axis=1, keepdims=True)
    fn1 = jnp.sum(xt * bn[1:2, :], axis=1, keepdims=True)
    pn = fn0 * fn1

    an = an_ref[0]
    sp = jax.lax.broadcasted_iota(jnp.int32, (_NUM_SPECIES, an.shape[1]), 0)
    onehot = (sp == an).astype(jnp.bfloat16)
    acc[:, :_KH] += _dot(onehot, pre)
    acc[:, _KH:] += _dot(onehot, pim)
    nyq[...] += _dot(onehot.astype(jnp.float32), pn)
    counts[...] += jnp.sum(onehot.astype(jnp.float32), axis=1, keepdims=True)

    @pl.when(i == nsteps - 1)
    def _fini():
        k = jax.lax.broadcasted_iota(jnp.int32, (_NUM_SPECIES, _KH), 1)
        w = jnp.where(k == 0, 1.0, 2.0).astype(jnp.float32)
        are = acc[:, :_KH] * w
        aim = acc[:, _KH:] * w
        t = jax.lax.broadcasted_iota(jnp.int32, (1, _NRF), 1)
        alt_t = jnp.where(t % 2 == 0, 1.0, -1.0).astype(jnp.float32)
        z = (_dot(are, ecf_ref[...], jax.lax.Precision.HIGHEST)
             - _dot(aim, esf_ref[...], jax.lax.Precision.HIGHEST)
             + nyq[:, 0:1] * alt_t)
        denom = (counts[:, 0:1] + 1.0) * float(_NUM_SPECIES * _NRF)
        out_ref[...] = z / denom


@jax.jit
def _run(x, an3, count_sketches, ecb, esb, ecf, esf):
    n = x.shape[0]
    nsteps = n // _TILE
    out = pl.pallas_call(
        _sketch_kernel,
        grid=(nsteps,),
        in_specs=[
            pl.BlockSpec((_TILE, _IN_DIM), lambda i: (i, 0)),
            pl.BlockSpec((1, 1, _TILE), lambda i: (i, 0, 0)),
            pl.BlockSpec((_DEGREE, _NRF, _IN_DIM), lambda i: (0, 0, 0)),
            pl.BlockSpec((_NRF, _KH), lambda i: (0, 0)),
            pl.BlockSpec((_NRF, _KH), lambda i: (0, 0)),
            pl.BlockSpec((_KH, _NRF), lambda i: (0, 0)),
            pl.BlockSpec((_KH, _NRF), lambda i: (0, 0)),
        ],
        out_specs=pl.BlockSpec((_NUM_SPECIES, _NRF), lambda i: (0, 0)),
        out_shape=jax.ShapeDtypeStruct((_NUM_SPECIES, _NRF), jnp.float32),
        scratch_shapes=[
            pltpu.VMEM((_IN_DIM, _KH), jnp.bfloat16),
            pltpu.VMEM((_IN_DIM, _KH), jnp.bfloat16),
            pltpu.VMEM((_IN_DIM, _KH), jnp.bfloat16),
            pltpu.VMEM((_IN_DIM, _KH), jnp.bfloat16),
            pltpu.VMEM((_DEGREE, _IN_DIM), jnp.float32),
            pltpu.VMEM((_NUM_SPECIES, _NRF), jnp.float32),
            pltpu.VMEM((_NUM_SPECIES, 128), jnp.float32),
            pltpu.VMEM((_NUM_SPECIES, 128), jnp.float32),
        ],
    )(x, an3, count_sketches, ecb, esb, ecf, esf)
    return out.reshape(-1)


def kernel(x, atomic_numbers, count_sketches):
    n = x.shape[0]
    r = jnp.arange(_NRF, dtype=jnp.int32)
    m = (r[:, None] * r[None, :]) % _NRF
    ang = m.astype(jnp.float32) * (2.0 * jnp.pi / _NRF)
    ecos = jnp.cos(ang)
    esin = jnp.sin(ang)
    ecb, esb = ecos[:, :_KH], esin[:, :_KH]
    ecf, esf = ecos[:_KH, :], esin[:_KH, :]
    an3 = atomic_numbers.astype(jnp.int32).reshape(n // _TILE, 1, _TILE)
    return _run(x, an3, count_sketches, ecb, esb, ecf, esf)


# speedup vs baseline: 123.0650x; 1.2117x over previous
import functools

import jax
import jax.numpy as jnp
import numpy as np
from jax.experimental import pallas as pl
from jax.experimental.pallas import tpu as pltpu

_DEGREE = 2
_NRF = 512
_KH = _NRF // 2
_IN_DIM = 128
_NUM_SPECIES = 8
_LENGTH_SCALE = 1.0
_TILE = 5000


def _dot(a, b, precision=jax.lax.Precision.DEFAULT,
         out_dtype=jnp.float32):
    return jax.lax.dot_general(
        a, b, (((1,), (0,)), ((), ())),
        precision=precision,
        preferred_element_type=out_dtype,
    )


def _dot_c00(a, b):
    return jax.lax.dot_general(
        a, b, (((0,), (0,)), ((), ())),
        precision=jax.lax.Precision.HIGHEST,
        preferred_element_type=jnp.float32,
    )


def _sketch_kernel(x_ref, an_ref, cs_ref, ecb_ref, esb_ref, ecf_ref, esf_ref,
                   out_ref,
                   b0re, b0im, b1re, b1im, bn, acc, nyq, counts):
    i = pl.program_id(0)
    nsteps = pl.num_programs(0)

    @pl.when(i == 0)
    def _init():
        b0re[...] = _dot_c00(cs_ref[0], ecb_ref[...]).astype(jnp.bfloat16)
        b0im[...] = (-_dot_c00(cs_ref[0], esb_ref[...])).astype(jnp.bfloat16)
        b1re[...] = _dot_c00(cs_ref[1], ecb_ref[...]).astype(jnp.bfloat16)
        b1im[...] = (-_dot_c00(cs_ref[1], esb_ref[...])).astype(jnp.bfloat16)
        r = jax.lax.broadcasted_iota(jnp.int32, (_NRF, 1), 0)
        alt = jnp.where(r % 2 == 0, 1.0, -1.0).astype(jnp.float32)
        bn[...] = jnp.concatenate(
            [jnp.sum(cs_ref[0] * alt, axis=0, keepdims=True),
             jnp.sum(cs_ref[1] * alt, axis=0, keepdims=True)], axis=0)
        acc[...] = jnp.zeros_like(acc)
        nyq[...] = jnp.zeros_like(nyq)
        counts[...] = jnp.zeros_like(counts)

    xt = x_ref[...] * (1.0 / _LENGTH_SCALE)
    xb = xt.astype(jnp.bfloat16)
    f0re = _dot(xb, b0re[...])
    f0im = _dot(xb, b0im[...])
    f1re = _dot(xb, b1re[...])
    f1im = _dot(xb, b1im[...])

    pre = (f0re * f1re - f0im * f1im).astype(jnp.bfloat16)
    pim = (f0re * f1im + f0im * f1re).astype(jnp.bfloat16)

    fn0 = jnp.sum(xt * bn[0:1, :], axis=1, keepdims=True)
    fn1 = jnp.sum(xt * bn[1:2, :], axis=1, keepdims=True)
    pn = fn0 * fn1

    an = an_ref[0]
    sp = jax.lax.broadcasted_iota(jnp.int32, (_NUM_SPECIES, an.shape[1]), 0)
    onehot = (sp == an).astype(jnp.bfloat16)
    acc[:, :_KH] += _dot(onehot, pre)
    acc[:, _KH:] += _dot(onehot, pim)
    nyq[...] += _dot(onehot.astype(jnp.float32), pn)
    counts[...] += jnp.sum(onehot.astype(jnp.float32), axis=1, keepdims=True)

    @pl.when(i == nsteps - 1)
    def _fini():
        k = jax.lax.broadcasted_iota(jnp.int32, (_NUM_SPECIES, _KH), 1)
        w = jnp.where(k == 0, 1.0, 2.0).astype(jnp.float32)
        are = acc[:, :_KH] * w
        aim = acc[:, _KH:] * w
        t = jax.lax.broadcasted_iota(jnp.int32, (1, _NRF), 1)
        alt_t = jnp.where(t % 2 == 0, 1.0, -1.0).astype(jnp.float32)
        z = (_dot(are, ecf_ref[...], jax.lax.Precision.HIGHEST)
             - _dot(aim, esf_ref[...], jax.lax.Precision.HIGHEST)
             + nyq[:, 0:1] * alt_t)
        denom = (counts[:, 0:1] + 1.0) * float(_NUM_SPECIES * _NRF)
        out_ref[...] = z / denom


@jax.jit
def _run(x, an3, count_sketches, ecb, esb, ecf, esf):
    n = x.shape[0]
    nsteps = n // _TILE
    out = pl.pallas_call(
        _sketch_kernel,
        grid=(nsteps,),
        in_specs=[
            pl.BlockSpec((_TILE, _IN_DIM), lambda i: (i, 0)),
            pl.BlockSpec((1, 1, _TILE), lambda i: (i, 0, 0)),
            pl.BlockSpec((_DEGREE, _NRF, _IN_DIM), lambda i: (0, 0, 0)),
            pl.BlockSpec((_NRF, _KH), lambda i: (0, 0)),
            pl.BlockSpec((_NRF, _KH), lambda i: (0, 0)),
            pl.BlockSpec((_KH, _NRF), lambda i: (0, 0)),
            pl.BlockSpec((_KH, _NRF), lambda i: (0, 0)),
        ],
        out_specs=pl.BlockSpec((_NUM_SPECIES, _NRF), lambda i: (0, 0)),
        out_shape=jax.ShapeDtypeStruct((_NUM_SPECIES, _NRF), jnp.float32),
        scratch_shapes=[
            pltpu.VMEM((_IN_DIM, _KH), jnp.bfloat16),
            pltpu.VMEM((_IN_DIM, _KH), jnp.bfloat16),
            pltpu.VMEM((_IN_DIM, _KH), jnp.bfloat16),
            pltpu.VMEM((_IN_DIM, _KH), jnp.bfloat16),
            pltpu.VMEM((_DEGREE, _IN_DIM), jnp.float32),
            pltpu.VMEM((_NUM_SPECIES, _NRF), jnp.float32),
            pltpu.VMEM((_NUM_SPECIES, 128), jnp.float32),
            pltpu.VMEM((_NUM_SPECIES, 128), jnp.float32),
        ],
    )(x, an3, count_sketches, ecb, esb, ecf, esf)
    return out.reshape(-1)


_R = np.arange(_NRF)
_ANG = ((_R[:, None] * _R[None, :]) % _NRF).astype(np.float64) * (2.0 * np.pi / _NRF)
_ECOS = np.cos(_ANG).astype(np.float32)
_ESIN = np.sin(_ANG).astype(np.float32)
_ECB, _ESB = jnp.asarray(_ECOS[:, :_KH]), jnp.asarray(_ESIN[:, :_KH])
_ECF, _ESF = jnp.asarray(_ECOS[:_KH, :]), jnp.asarray(_ESIN[:_KH, :])


def kernel(x, atomic_numbers, count_sketches):
    n = x.shape[0]
    an3 = atomic_numbers.astype(jnp.int32).reshape(n // _TILE, 1, _TILE)
    return _run(x, an3, count_sketches, _ECB, _ESB, _ECF, _ESF)


# speedup vs baseline: 124.2200x; 1.0094x over previous
import functools

import jax
import jax.numpy as jnp
import numpy as np
from jax.experimental import pallas as pl
from jax.experimental.pallas import tpu as pltpu

_DEGREE = 2
_NRF = 512
_KH = _NRF // 2
_IN_DIM = 128
_NUM_SPECIES = 8
_LENGTH_SCALE = 1.0
_TILE = 5000


def _dot(a, b, precision=jax.lax.Precision.DEFAULT,
         out_dtype=jnp.float32):
    return jax.lax.dot_general(
        a, b, (((1,), (0,)), ((), ())),
        precision=precision,
        preferred_element_type=out_dtype,
    )


def _dot_c00(a, b):
    return jax.lax.dot_general(
        a, b, (((0,), (0,)), ((), ())),
        precision=jax.lax.Precision.HIGHEST,
        preferred_element_type=jnp.float32,
    )


def _sketch_kernel(x_ref, an_ref, cs_ref, ecb_ref, esb_ref, ecf_ref, esf_ref,
                   out_ref,
                   b0re, b0im, b1re, b1im, bn, acc, nc):
    i = pl.program_id(0)
    nsteps = pl.num_programs(0)

    @pl.when(i == 0)
    def _init():
        b0re[...] = _dot_c00(cs_ref[0], ecb_ref[...]).astype(jnp.bfloat16)
        b0im[...] = (-_dot_c00(cs_ref[0], esb_ref[...])).astype(jnp.bfloat16)
        b1re[...] = _dot_c00(cs_ref[1], ecb_ref[...]).astype(jnp.bfloat16)
        b1im[...] = (-_dot_c00(cs_ref[1], esb_ref[...])).astype(jnp.bfloat16)
        r = jax.lax.broadcasted_iota(jnp.int32, (_NRF, 1), 0)
        alt = jnp.where(r % 2 == 0, 1.0, -1.0).astype(jnp.float32)
        bn[...] = jnp.concatenate(
            [jnp.sum(cs_ref[0] * alt, axis=0, keepdims=True),
             jnp.sum(cs_ref[1] * alt, axis=0, keepdims=True)], axis=0)
        acc[...] = jnp.zeros_like(acc)
        nc[...] = jnp.zeros_like(nc)

    xt = x_ref[...]
    if _LENGTH_SCALE != 1.0:
        xt = xt * (1.0 / _LENGTH_SCALE)
    xb = xt.astype(jnp.bfloat16)
    f0re = _dot(xb, b0re[...])
    f0im = _dot(xb, b0im[...])
    f1re = _dot(xb, b1re[...])
    f1im = _dot(xb, b1im[...])

    pre = (f0re * f1re - f0im * f1im).astype(jnp.bfloat16)
    pim = (f0re * f1im + f0im * f1re).astype(jnp.bfloat16)

    fn0 = jnp.sum(xt * bn[0:1, :], axis=1, keepdims=True)
    fn1 = jnp.sum(xt * bn[1:2, :], axis=1, keepdims=True)
    pn = (fn0 * fn1).astype(jnp.bfloat16)
    pn1 = jnp.concatenate([pn, jnp.ones_like(pn)], axis=1)

    an = an_ref[0]
    sp = jax.lax.broadcasted_iota(jnp.int32, (_NUM_SPECIES, an.shape[1]), 0)
    onehot = (sp == an).astype(jnp.bfloat16)
    acc[:, :_KH] += _dot(onehot, pre)
    acc[:, _KH:] += _dot(onehot, pim)
    nc[...] += _dot(onehot, pn1)

    @pl.when(i == nsteps - 1)
    def _fini():
        k = jax.lax.broadcasted_iota(jnp.int32, (_NUM_SPECIES, _KH), 1)
        w = jnp.where(k == 0, 1.0, 2.0).astype(jnp.float32)
        are = acc[:, :_KH] * w
        aim = acc[:, _KH:] * w
        t = jax.lax.broadcasted_iota(jnp.int32, (1, _NRF), 1)
        alt_t = jnp.where(t % 2 == 0, 1.0, -1.0).astype(jnp.float32)
        z = (_dot(are, ecf_ref[...], jax.lax.Precision.HIGHEST)
             - _dot(aim, esf_ref[...], jax.lax.Precision.HIGHEST)
             + nc[:, 0:1] * alt_t)
        denom = (nc[:, 1:2] + 1.0) * float(_NUM_SPECIES * _NRF)
        out_ref[...] = z / denom


@jax.jit
def _run(x, an3, count_sketches, ecb, esb, ecf, esf):
    n = x.shape[0]
    nsteps = n // _TILE
    out = pl.pallas_call(
        _sketch_kernel,
        grid=(nsteps,),
        in_specs=[
            pl.BlockSpec((_TILE, _IN_DIM), lambda i: (i, 0)),
            pl.BlockSpec((1, 1, _TILE), lambda i: (i, 0, 0)),
            pl.BlockSpec((_DEGREE, _NRF, _IN_DIM), lambda i: (0, 0, 0)),
            pl.BlockSpec((_NRF, _KH), lambda i: (0, 0)),
            pl.BlockSpec((_NRF, _KH), lambda i: (0, 0)),
            pl.BlockSpec((_KH, _NRF), lambda i: (0, 0)),
            pl.BlockSpec((_KH, _NRF), lambda i: (0, 0)),
        ],
        out_specs=pl.BlockSpec((_NUM_SPECIES, _NRF), lambda i: (0, 0)),
        out_shape=jax.ShapeDtypeStruct((_NUM_SPECIES, _NRF), jnp.float32),
        scratch_shapes=[
            pltpu.VMEM((_IN_DIM, _KH), jnp.bfloat16),
            pltpu.VMEM((_IN_DIM, _KH), jnp.bfloat16),
            pltpu.VMEM((_IN_DIM, _KH), jnp.bfloat16),
            pltpu.VMEM((_IN_DIM, _KH), jnp.bfloat16),
            pltpu.VMEM((_DEGREE, _IN_DIM), jnp.float32),
            pltpu.VMEM((_NUM_SPECIES, _NRF), jnp.float32),
            pltpu.VMEM((_NUM_SPECIES, 2), jnp.float32),
        ],
    )(x, an3, count_sketches, ecb, esb, ecf, esf)
    return out.reshape(-1)


_R = np.arange(_NRF)
_ANG = ((_R[:, None] * _R[None, :]) % _NRF).astype(np.float64) * (2.0 * np.pi / _NRF)
_ECOS = np.cos(_ANG).astype(np.float32)
_ESIN = np.sin(_ANG).astype(np.float32)
_ECB, _ESB = jnp.asarray(_ECOS[:, :_KH]), jnp.asarray(_ESIN[:, :_KH])
_ECF, _ESF = jnp.asarray(_ECOS[:_KH, :]), jnp.asarray(_ESIN[:_KH, :])


def kernel(x, atomic_numbers, count_sketches):
    n = x.shape[0]
    an3 = atomic_numbers.astype(jnp.int32).reshape(n // _TILE, 1, _TILE)
    return _run(x, an3, count_sketches, _ECB, _ESB, _ECF, _ESF)


# speedup vs baseline: 126.5818x; 1.0190x over previous
import functools

import jax
import jax.numpy as jnp
import numpy as np
from jax.experimental import pallas as pl
from jax.experimental.pallas import tpu as pltpu

_DEGREE = 2
_NRF = 512
_KH = _NRF // 2
_IN_DIM = 128
_NUM_SPECIES = 8
_LENGTH_SCALE = 1.0
_TILE = 5000


def _dot(a, b, precision=jax.lax.Precision.DEFAULT,
         out_dtype=jnp.float32):
    return jax.lax.dot_general(
        a, b, (((1,), (0,)), ((), ())),
        precision=precision,
        preferred_element_type=out_dtype,
    )


def _dot_c00(a, b):
    return jax.lax.dot_general(
        a, b, (((0,), (0,)), ((), ())),
        precision=jax.lax.Precision.DEFAULT,
        preferred_element_type=jnp.float32,
    )


def _sketch_kernel(x_ref, an_ref, cs_ref, ecb_ref, esb_ref, ecf_ref, esf_ref,
                   out_ref,
                   b0re, b0im, b1re, b1im, bn, acc, nc):
    i = pl.program_id(0)
    nsteps = pl.num_programs(0)

    @pl.when(i == 0)
    def _init():
        b0re[...] = _dot_c00(cs_ref[0], ecb_ref[...]).astype(jnp.bfloat16)
        b0im[...] = (-_dot_c00(cs_ref[0], esb_ref[...])).astype(jnp.bfloat16)
        b1re[...] = _dot_c00(cs_ref[1], ecb_ref[...]).astype(jnp.bfloat16)
        b1im[...] = (-_dot_c00(cs_ref[1], esb_ref[...])).astype(jnp.bfloat16)
        r = jax.lax.broadcasted_iota(jnp.int32, (_NRF, 1), 0)
        alt = jnp.where(r % 2 == 0, 1.0, -1.0).astype(jnp.float32)
        bn[...] = jnp.concatenate(
            [jnp.sum(cs_ref[0] * alt, axis=0, keepdims=True),
             jnp.sum(cs_ref[1] * alt, axis=0, keepdims=True)], axis=0)
        acc[...] = jnp.zeros_like(acc)
        nc[...] = jnp.zeros_like(nc)

    xt = x_ref[...]
    if _LENGTH_SCALE != 1.0:
        xt = xt * (1.0 / _LENGTH_SCALE)
    xb = xt.astype(jnp.bfloat16)
    f0re = _dot(xb, b0re[...])
    f0im = _dot(xb, b0im[...])
    f1re = _dot(xb, b1re[...])
    f1im = _dot(xb, b1im[...])

    pre = (f0re * f1re - f0im * f1im).astype(jnp.bfloat16)
    pim = (f0re * f1im + f0im * f1re).astype(jnp.bfloat16)

    fn0 = jnp.sum(xt * bn[0:1, :], axis=1, keepdims=True)
    fn1 = jnp.sum(xt * bn[1:2, :], axis=1, keepdims=True)
    pn = (fn0 * fn1).astype(jnp.bfloat16)
    pn1 = jnp.concatenate([pn, jnp.ones_like(pn)], axis=1)

    an = an_ref[0]
    sp = jax.lax.broadcasted_iota(jnp.int32, (_NUM_SPECIES, an.shape[1]), 0)
    onehot = (sp == an).astype(jnp.bfloat16)
    acc[:, :_KH] += _dot(onehot, pre)
    acc[:, _KH:] += _dot(onehot, pim)
    nc[...] += _dot(onehot, pn1)

    @pl.when(i == nsteps - 1)
    def _fini():
        k = jax.lax.broadcasted_iota(jnp.int32, (_NUM_SPECIES, _KH), 1)
        w = jnp.where(k == 0, 1.0, 2.0).astype(jnp.float32)
        are = acc[:, :_KH] * w
        aim = acc[:, _KH:] * w
        t = jax.lax.broadcasted_iota(jnp.int32, (1, _NRF), 1)
        alt_t = jnp.where(t % 2 == 0, 1.0, -1.0).astype(jnp.float32)
        z = (_dot(are, ecf_ref[...], jax.lax.Precision.HIGHEST)
             - _dot(aim, esf_ref[...], jax.lax.Precision.HIGHEST)
             + nc[:, 0:1] * alt_t)
        denom = (nc[:, 1:2] + 1.0) * float(_NUM_SPECIES * _NRF)
        out_ref[...] = z / denom


@jax.jit
def _run(x, an3, count_sketches, ecb, esb, ecf, esf):
    n = x.shape[0]
    nsteps = n // _TILE
    out = pl.pallas_call(
        _sketch_kernel,
        grid=(nsteps,),
        in_specs=[
            pl.BlockSpec((_TILE, _IN_DIM), lambda i: (i, 0)),
            pl.BlockSpec((1, 1, _TILE), lambda i: (i, 0, 0)),
            pl.BlockSpec((_DEGREE, _NRF, _IN_DIM), lambda i: (0, 0, 0)),
            pl.BlockSpec((_NRF, _KH), lambda i: (0, 0)),
            pl.BlockSpec((_NRF, _KH), lambda i: (0, 0)),
            pl.BlockSpec((_KH, _NRF), lambda i: (0, 0)),
            pl.BlockSpec((_KH, _NRF), lambda i: (0, 0)),
        ],
        out_specs=pl.BlockSpec((_NUM_SPECIES, _NRF), lambda i: (0, 0)),
        out_shape=jax.ShapeDtypeStruct((_NUM_SPECIES, _NRF), jnp.float32),
        scratch_shapes=[
            pltpu.VMEM((_IN_DIM, _KH), jnp.bfloat16),
            pltpu.VMEM((_IN_DIM, _KH), jnp.bfloat16),
            pltpu.VMEM((_IN_DIM, _KH), jnp.bfloat16),
            pltpu.VMEM((_IN_DIM, _KH), jnp.bfloat16),
            pltpu.VMEM((_DEGREE, _IN_DIM), jnp.float32),
            pltpu.VMEM((_NUM_SPECIES, _NRF), jnp.float32),
            pltpu.VMEM((_NUM_SPECIES, 2), jnp.float32),
        ],
    )(x, an3, count_sketches, ecb, esb, ecf, esf)
    return out.reshape(-1)


_R = np.arange(_NRF)
_ANG = ((_R[:, None] * _R[None, :]) % _NRF).astype(np.float64) * (2.0 * np.pi / _NRF)
_ECOS = np.cos(_ANG).astype(np.float32)
_ESIN = np.sin(_ANG).astype(np.float32)
_ECB, _ESB = jnp.asarray(_ECOS[:, :_KH]), jnp.asarray(_ESIN[:, :_KH])
_ECF, _ESF = jnp.asarray(_ECOS[:_KH, :]), jnp.asarray(_ESIN[:_KH, :])


def kernel(x, atomic_numbers, count_sketches):
    n = x.shape[0]
    an3 = atomic_numbers.astype(jnp.int32).reshape(n // _TILE, 1, _TILE)
    return _run(x, an3, count_sketches, _ECB, _ESB, _ECF, _ESF)
